# Initial kernel scaffold; baseline (speedup 1.0000x reference)
import jax
import jax.numpy as jnp
from jax.experimental import pallas as pl


def kernel(seg_masks_soft, cate_scores, cate_labels):
    raise NotImplementedError("write your pallas kernel here")


# speedup vs baseline: 4.6902x; 4.6902x over previous
import jax
import jax.numpy as jnp
from jax.experimental import pallas as pl

N = 1000
K = 104 * 104
BK = 2176
NKB = 5
MASK_THR = 0.5
SIGMA = 2.0


def _gram_kernel(soft_ref, g_ref):
    kb = pl.program_id(0)
    x = soft_ref[...]
    kcol = jax.lax.broadcasted_iota(jnp.int32, (N, BK), 1) + kb * BK
    b = ((x > MASK_THR) & (kcol < K)).astype(jnp.bfloat16)
    part = jax.lax.dot_general(
        b, b, (((1,), (1,)), ((), ())), preferred_element_type=jnp.float32)

    @pl.when(kb == 0)
    def _():
        g_ref[...] = part

    @pl.when(kb != 0)
    def _():
        g_ref[...] += part


def _nms_kernel(g_ref, sr_ref, sc_ref, lr_ref, lc_ref, out_ref):
    g = g_ref[...]
    sr = sr_ref[...]
    sc = sc_ref[...]
    lr = lr_ref[...]
    lc = lc_ref[...]
    iu = jax.lax.broadcasted_iota(jnp.int32, (N, N), 0)
    iv = jax.lax.broadcasted_iota(jnp.int32, (N, N), 1)

    diag = iu == iv
    s_col = jnp.sum(jnp.where(diag, g, 0.0), axis=1, keepdims=True)
    s_row = jnp.sum(jnp.where(diag, g, 0.0), axis=0, keepdims=True)

    den = s_col + s_row - g
    iou = jnp.where(den > 0.0, g, 0.0) / jnp.where(den > 0.0, den, 1.0)

    order = (sc > sr) | ((sc == sr) & (iu < iv))
    ordt = (sr > sc) | ((sr == sc) & (iv < iu))
    lbl = lc == lr

    m = jnp.where(order & lbl, iou, 0.0)
    mt = jnp.where(ordt & lbl, iou, 0.0)

    c_row = jnp.max(m, axis=0, keepdims=True)
    decay_t = jnp.exp(-SIGMA * mt * mt)
    comp_r = jnp.exp(-SIGMA * c_row * c_row)
    d_col = jnp.min(decay_t / comp_r, axis=1, keepdims=True)

    val_col = sc * d_col
    rank_col = jnp.sum(ordt.astype(jnp.float32), axis=1, keepdims=True)
    onehot = rank_col == iv.astype(jnp.float32)
    out_ref[...] = jnp.sum(jnp.where(onehot, val_col, 0.0),
                           axis=0, keepdims=True)


def kernel(seg_masks_soft, cate_scores, cate_labels):
    soft = seg_masks_soft.reshape(N, K)
    g = pl.pallas_call(
        _gram_kernel,
        grid=(NKB,),
        in_specs=[pl.BlockSpec((N, BK), lambda kb: (0, kb))],
        out_specs=pl.BlockSpec((N, N), lambda kb: (0, 0)),
        out_shape=jax.ShapeDtypeStruct((N, N), jnp.float32),
    )(soft)

    sr = cate_scores.reshape(1, N)
    sc = cate_scores.reshape(N, 1)
    lr = cate_labels.reshape(1, N)
    lc = cate_labels.reshape(N, 1)
    out = pl.pallas_call(
        _nms_kernel,
        out_shape=jax.ShapeDtypeStruct((1, N), jnp.float32),
    )(g, sr, sc, lr, lc)
    return out.reshape(N)
